# baseline (speedup 1.0000x reference)
import functools

import jax
import jax.numpy as jnp
from jax import lax
from jax.experimental import pallas as pl
from jax.experimental.pallas import tpu as pltpu
from jax.experimental.pallas import tpu_sc as plsc

BATCH = 4096
HIST = 200
EMB = 64
NCLS = 100
VOCAB = 1000000

NC = 2
NS = 16
NW = NC * NS

RBLK = 4096
RSH = 12
RGRID = (VOCAB + RBLK - 1) // RBLK
TROWS = RGRID * (RBLK // 2)

B_PER_W = BATCH // NW
NBUF = 2
SPLIT0 = 104
SPLIT1 = HIST - SPLIT0


def _relayout_body(i_ref, o_ref):
    t = i_ref[...].T
    o_ref[:, 0:EMB] = t[0:RBLK // 2, :]
    o_ref[:, EMB:2 * EMB] = t[RBLK // 2:, :]


_tc_relayout = pl.pallas_call(
    _relayout_body,
    grid=(RGRID,),
    in_specs=[pl.BlockSpec((EMB, RBLK), lambda i: (0, i))],
    out_specs=pl.BlockSpec((RBLK // 2, 2 * EMB), lambda i: (i, 0)),
    out_shape=jax.ShapeDtypeStruct((TROWS, 2 * EMB), jnp.float32),
)


def _pool_kernel(x_hbm, tab_hbm, out_hbm,
                 jdx_v, pof_v, g0, g1, out_v, s0, s1):
    gbufs = (g0, g1)
    sems = (s0, s1)
    wid = lax.axis_index("s") * NC + lax.axis_index("c")
    base = wid * B_PER_W
    iota = lax.iota(jnp.int32, 16)

    pltpu.sync_copy(x_hbm.at[pl.ds(base * HIST, B_PER_W * HIST)], jdx_v)

    def prep(i, carry):
        raw = jdx_v[pl.ds(i * 16, 16)]
        pof_v[pl.ds(i * 16, 16)] = ((raw >> (RSH - 1)) & 1) * EMB
        jdx_v[pl.ds(i * 16, 16)] = ((raw >> RSH) << (RSH - 1)) + \
            (raw & (RBLK // 2 - 1))
        return carry
    lax.fori_loop(0, B_PER_W * HIST // 16, prep, 0)

    def fire(b, slot):
        off = b * HIST
        pltpu.async_copy(tab_hbm.at[jdx_v.at[pl.ds(off, SPLIT0)]],
                         gbufs[slot].at[pl.ds(0, SPLIT0)], sems[slot])
        pltpu.async_copy(tab_hbm.at[jdx_v.at[pl.ds(off + SPLIT0, SPLIT1)]],
                         gbufs[slot].at[pl.ds(SPLIT0, SPLIT1)], sems[slot])

    def drain(slot):
        pltpu.make_async_copy(tab_hbm.at[pl.ds(0, HIST)], gbufs[slot],
                              sems[slot]).wait()

    def consume(b, slot):
        gb = gbufs[slot]
        off = b * HIST

        def rows(l, acc):
            a0, a1, a2, a3 = acc
            p = plsc.load_gather(pof_v, [jnp.full((16,), off + l, jnp.int32)])
            lsplat = jnp.full((16,), l, jnp.int32)
            col = p + iota
            a0 = a0 + plsc.load_gather(gb, [lsplat, col])
            a1 = a1 + plsc.load_gather(gb, [lsplat, col + 16])
            a2 = a2 + plsc.load_gather(gb, [lsplat, col + 32])
            a3 = a3 + plsc.load_gather(gb, [lsplat, col + 48])
            return (a0, a1, a2, a3)
        zero = jnp.zeros((16,), jnp.float32)
        acc = lax.fori_loop(0, HIST, rows, (zero, zero, zero, zero))

        for c in range(EMB // 16):
            out_v[pl.ds(b * EMB + c * 16, 16)] = acc[c]

    fire(0, 0)

    def group(g, carry):
        for s in range(NBUF):
            b = g * NBUF + s
            nb = b + NBUF - 1
            nslot = (s + NBUF - 1) % NBUF

            @pl.when(nb < B_PER_W)
            def _():
                fire(nb, nslot)

            drain(s)
            consume(b, s)
        return carry
    lax.fori_loop(0, B_PER_W // NBUF, group, 0)

    pltpu.sync_copy(out_v, out_hbm.at[pl.ds(base * EMB, B_PER_W * EMB)])


_pool = functools.partial(
    pl.kernel,
    out_type=jax.ShapeDtypeStruct((BATCH * EMB,), jnp.float32),
    mesh=plsc.VectorSubcoreMesh(core_axis_name="c", subcore_axis_name="s"),
    compiler_params=pltpu.CompilerParams(needs_layout_passes=False),
    scratch_types=[
        pltpu.VMEM((B_PER_W * HIST,), jnp.int32),
        pltpu.VMEM((B_PER_W * HIST,), jnp.int32),
        pltpu.VMEM((HIST, 2 * EMB), jnp.float32),
        pltpu.VMEM((HIST, 2 * EMB), jnp.float32),
        pltpu.VMEM((B_PER_W * EMB,), jnp.float32),
        pltpu.SemaphoreType.DMA,
        pltpu.SemaphoreType.DMA,
    ],
)(_pool_kernel)


def _mm_body(m_ref, x_ref, e0_ref, w_ref, b_ref, o_ref):
    mm = lax.dot_general(
        m_ref[...], w_ref[...], (((1,), (1,)), ((), ())),
        preferred_element_type=jnp.float32)
    e0w = lax.dot_general(
        e0_ref[...], w_ref[...], (((1,), (1,)), ((), ())),
        preferred_element_type=jnp.float32)
    cnt = jnp.sum((x_ref[...] == 0).astype(jnp.float32), axis=1,
                  keepdims=True)
    o_ref[...] = (mm - cnt * e0w) * (1.0 / HIST) + b_ref[...]


def _classify(m, x, e0, W, b):
    return pl.pallas_call(
        _mm_body,
        out_shape=jax.ShapeDtypeStruct((BATCH, NCLS), jnp.float32),
    )(m, x, e0, W, b.reshape(1, NCLS))


def kernel(x, emb, W, b):
    tab = _tc_relayout(emb.T)
    pooled = _pool(x.reshape(-1), tab)
    m = pooled.reshape(BATCH, EMB)
    return _classify(m, x, emb[0:1, :], W, b)


# speedup vs baseline: 1.1340x; 1.1340x over previous
import functools

import jax
import jax.numpy as jnp
from jax import lax
from jax.experimental import pallas as pl
from jax.experimental.pallas import tpu as pltpu
from jax.experimental.pallas import tpu_sc as plsc

BATCH = 4096
HIST = 200
EMB = 64
NCLS = 100
VOCAB = 1000000

NC = 2
NS = 16
NW = NC * NS

RBLK = 8192
RSH = 13
RGRID = (VOCAB + RBLK - 1) // RBLK
TROWS = RGRID * (RBLK // 2)

B_PER_W = BATCH // NW
NBUF = 2
SPLIT0 = 104
SPLIT1 = HIST - SPLIT0


def _relayout_body(i_ref, o_ref):
    t = i_ref[...].T
    o_ref[:, 0:EMB] = t[0:RBLK // 2, :]
    o_ref[:, EMB:2 * EMB] = t[RBLK // 2:, :]


_tc_relayout = pl.pallas_call(
    _relayout_body,
    grid=(RGRID,),
    in_specs=[pl.BlockSpec((EMB, RBLK), lambda i: (0, i))],
    out_specs=pl.BlockSpec((RBLK // 2, 2 * EMB), lambda i: (i, 0)),
    out_shape=jax.ShapeDtypeStruct((TROWS, 2 * EMB), jnp.float32),
)


def _pool_kernel(x_hbm, tab_hbm, out_hbm,
                 jdx_v, pof_v, g0, g1, out_v, s0, s1):
    gbufs = (g0, g1)
    sems = (s0, s1)
    wid = lax.axis_index("s") * NC + lax.axis_index("c")
    base = wid * B_PER_W
    iota = lax.iota(jnp.int32, 16)

    pltpu.sync_copy(x_hbm.at[pl.ds(base * HIST, B_PER_W * HIST)], jdx_v)

    def prep(i, carry):
        raw = jdx_v[pl.ds(i * 16, 16)]
        pof_v[pl.ds(i * 16, 16)] = ((raw >> (RSH - 1)) & 1) * EMB
        jdx_v[pl.ds(i * 16, 16)] = ((raw >> RSH) << (RSH - 1)) + \
            (raw & (RBLK // 2 - 1))
        return carry
    lax.fori_loop(0, B_PER_W * HIST // 16, prep, 0)

    def fire(b, slot):
        off = b * HIST
        pltpu.async_copy(tab_hbm.at[jdx_v.at[pl.ds(off, SPLIT0)]],
                         gbufs[slot].at[pl.ds(0, SPLIT0)], sems[slot])
        pltpu.async_copy(tab_hbm.at[jdx_v.at[pl.ds(off + SPLIT0, SPLIT1)]],
                         gbufs[slot].at[pl.ds(SPLIT0, SPLIT1)], sems[slot])

    def drain(slot):
        pltpu.make_async_copy(tab_hbm.at[pl.ds(0, HIST)], gbufs[slot],
                              sems[slot]).wait()

    def consume(b, slot):
        gb = gbufs[slot]
        off = b * HIST

        def rows(l, acc):
            a0, a1, a2, a3 = acc
            p = plsc.load_gather(pof_v, [jnp.full((16,), off + l, jnp.int32)])
            lsplat = jnp.full((16,), l, jnp.int32)
            col = p + iota
            a0 = a0 + plsc.load_gather(gb, [lsplat, col])
            a1 = a1 + plsc.load_gather(gb, [lsplat, col + 16])
            a2 = a2 + plsc.load_gather(gb, [lsplat, col + 32])
            a3 = a3 + plsc.load_gather(gb, [lsplat, col + 48])
            return (a0, a1, a2, a3)
        zero = jnp.zeros((16,), jnp.float32)
        acc = lax.fori_loop(0, HIST, rows, (zero, zero, zero, zero))

        for c in range(EMB // 16):
            out_v[pl.ds(b * EMB + c * 16, 16)] = acc[c]

    fire(0, 0)

    def group(g, carry):
        for s in range(NBUF):
            b = g * NBUF + s
            nb = b + NBUF - 1
            nslot = (s + NBUF - 1) % NBUF

            @pl.when(nb < B_PER_W)
            def _():
                fire(nb, nslot)

            drain(s)
            consume(b, s)
        return carry
    lax.fori_loop(0, B_PER_W // NBUF, group, 0)

    pltpu.sync_copy(out_v, out_hbm.at[pl.ds(base * EMB, B_PER_W * EMB)])


_pool = functools.partial(
    pl.kernel,
    out_type=jax.ShapeDtypeStruct((BATCH * EMB,), jnp.float32),
    mesh=plsc.VectorSubcoreMesh(core_axis_name="c", subcore_axis_name="s"),
    compiler_params=pltpu.CompilerParams(needs_layout_passes=False),
    scratch_types=[
        pltpu.VMEM((B_PER_W * HIST,), jnp.int32),
        pltpu.VMEM((B_PER_W * HIST,), jnp.int32),
        pltpu.VMEM((HIST, 2 * EMB), jnp.float32),
        pltpu.VMEM((HIST, 2 * EMB), jnp.float32),
        pltpu.VMEM((B_PER_W * EMB,), jnp.float32),
        pltpu.SemaphoreType.DMA,
        pltpu.SemaphoreType.DMA,
    ],
)(_pool_kernel)


def _mm_body(m_ref, x_ref, e0_ref, w_ref, b_ref, o_ref):
    mm = lax.dot_general(
        m_ref[...], w_ref[...], (((1,), (1,)), ((), ())),
        preferred_element_type=jnp.float32)
    e0w = lax.dot_general(
        e0_ref[...], w_ref[...], (((1,), (1,)), ((), ())),
        preferred_element_type=jnp.float32)
    cnt = jnp.sum((x_ref[...] == 0).astype(jnp.float32), axis=1,
                  keepdims=True)
    o_ref[...] = (mm - cnt * e0w) * (1.0 / HIST) + b_ref[...]


def _classify(m, x, e0, W, b):
    return pl.pallas_call(
        _mm_body,
        out_shape=jax.ShapeDtypeStruct((BATCH, NCLS), jnp.float32),
    )(m, x, e0, W, b.reshape(1, NCLS))


def kernel(x, emb, W, b):
    tab = _tc_relayout(emb.T)
    pooled = _pool(x.reshape(-1), tab)
    m = pooled.reshape(BATCH, EMB)
    return _classify(m, x, emb[0:1, :], W, b)


# speedup vs baseline: 1.2163x; 1.0726x over previous
import functools

import jax
import jax.numpy as jnp
from jax import lax
from jax.experimental import pallas as pl
from jax.experimental.pallas import tpu as pltpu
from jax.experimental.pallas import tpu_sc as plsc

BATCH = 4096
HIST = 200
EMB = 64
NCLS = 100
VOCAB = 1000000

NC = 2
NS = 16
NW = NC * NS

RBLK = 16384
RSH = 14
RGRID = (VOCAB + RBLK - 1) // RBLK
TROWS = RGRID * (RBLK // 2)

B_PER_W = BATCH // NW
NBUF = 2
SPLIT0 = 104
SPLIT1 = HIST - SPLIT0


def _relayout_body(i_ref, o_ref):
    t = i_ref[...].T
    o_ref[:, 0:EMB] = t[0:RBLK // 2, :]
    o_ref[:, EMB:2 * EMB] = t[RBLK // 2:, :]


_tc_relayout = pl.pallas_call(
    _relayout_body,
    grid=(RGRID,),
    in_specs=[pl.BlockSpec((EMB, RBLK), lambda i: (0, i))],
    out_specs=pl.BlockSpec((RBLK // 2, 2 * EMB), lambda i: (i, 0)),
    out_shape=jax.ShapeDtypeStruct((TROWS, 2 * EMB), jnp.float32),
)


def _pool_kernel(x_hbm, tab_hbm, out_hbm,
                 jdx_v, pof_v, g0, g1, out_v, s0, s1):
    gbufs = (g0, g1)
    sems = (s0, s1)
    wid = lax.axis_index("s") * NC + lax.axis_index("c")
    base = wid * B_PER_W
    iota = lax.iota(jnp.int32, 16)

    pltpu.sync_copy(x_hbm.at[pl.ds(base * HIST, B_PER_W * HIST)], jdx_v)

    def prep(i, carry):
        raw = jdx_v[pl.ds(i * 16, 16)]
        pof_v[pl.ds(i * 16, 16)] = ((raw >> (RSH - 1)) & 1) * EMB
        jdx_v[pl.ds(i * 16, 16)] = ((raw >> RSH) << (RSH - 1)) + \
            (raw & (RBLK // 2 - 1))
        return carry
    lax.fori_loop(0, B_PER_W * HIST // 16, prep, 0)

    def fire(b, slot):
        off = b * HIST
        pltpu.async_copy(tab_hbm.at[jdx_v.at[pl.ds(off, SPLIT0)]],
                         gbufs[slot].at[pl.ds(0, SPLIT0)], sems[slot])
        pltpu.async_copy(tab_hbm.at[jdx_v.at[pl.ds(off + SPLIT0, SPLIT1)]],
                         gbufs[slot].at[pl.ds(SPLIT0, SPLIT1)], sems[slot])

    def drain(slot):
        pltpu.make_async_copy(tab_hbm.at[pl.ds(0, HIST)], gbufs[slot],
                              sems[slot]).wait()

    def consume(b, slot):
        gb = gbufs[slot]
        off = b * HIST

        def rows(l, acc):
            a0, a1, a2, a3 = acc
            p = plsc.load_gather(pof_v, [jnp.full((16,), off + l, jnp.int32)])
            lsplat = jnp.full((16,), l, jnp.int32)
            col = p + iota
            a0 = a0 + plsc.load_gather(gb, [lsplat, col])
            a1 = a1 + plsc.load_gather(gb, [lsplat, col + 16])
            a2 = a2 + plsc.load_gather(gb, [lsplat, col + 32])
            a3 = a3 + plsc.load_gather(gb, [lsplat, col + 48])
            return (a0, a1, a2, a3)
        zero = jnp.zeros((16,), jnp.float32)
        acc = lax.fori_loop(0, HIST, rows, (zero, zero, zero, zero))

        for c in range(EMB // 16):
            out_v[pl.ds(b * EMB + c * 16, 16)] = acc[c]

    fire(0, 0)

    def group(g, carry):
        for s in range(NBUF):
            b = g * NBUF + s
            nb = b + NBUF - 1
            nslot = (s + NBUF - 1) % NBUF

            @pl.when(nb < B_PER_W)
            def _():
                fire(nb, nslot)

            drain(s)
            consume(b, s)
        return carry
    lax.fori_loop(0, B_PER_W // NBUF, group, 0)

    pltpu.sync_copy(out_v, out_hbm.at[pl.ds(base * EMB, B_PER_W * EMB)])


_pool = functools.partial(
    pl.kernel,
    out_type=jax.ShapeDtypeStruct((BATCH * EMB,), jnp.float32),
    mesh=plsc.VectorSubcoreMesh(core_axis_name="c", subcore_axis_name="s"),
    compiler_params=pltpu.CompilerParams(needs_layout_passes=False),
    scratch_types=[
        pltpu.VMEM((B_PER_W * HIST,), jnp.int32),
        pltpu.VMEM((B_PER_W * HIST,), jnp.int32),
        pltpu.VMEM((HIST, 2 * EMB), jnp.float32),
        pltpu.VMEM((HIST, 2 * EMB), jnp.float32),
        pltpu.VMEM((B_PER_W * EMB,), jnp.float32),
        pltpu.SemaphoreType.DMA,
        pltpu.SemaphoreType.DMA,
    ],
)(_pool_kernel)


def _mm_body(m_ref, x_ref, e0_ref, w_ref, b_ref, o_ref):
    mm = lax.dot_general(
        m_ref[...], w_ref[...], (((1,), (1,)), ((), ())),
        preferred_element_type=jnp.float32)
    e0w = lax.dot_general(
        e0_ref[...], w_ref[...], (((1,), (1,)), ((), ())),
        preferred_element_type=jnp.float32)
    cnt = jnp.sum((x_ref[...] == 0).astype(jnp.float32), axis=1,
                  keepdims=True)
    o_ref[...] = (mm - cnt * e0w) * (1.0 / HIST) + b_ref[...]


def _classify(m, x, e0, W, b):
    return pl.pallas_call(
        _mm_body,
        out_shape=jax.ShapeDtypeStruct((BATCH, NCLS), jnp.float32),
    )(m, x, e0, W, b.reshape(1, NCLS))


def kernel(x, emb, W, b):
    tab = _tc_relayout(emb.T)
    pooled = _pool(x.reshape(-1), tab)
    m = pooled.reshape(BATCH, EMB)
    return _classify(m, x, emb[0:1, :], W, b)


# speedup vs baseline: 1.2587x; 1.0349x over previous
import functools

import jax
import jax.numpy as jnp
from jax import lax
from jax.experimental import pallas as pl
from jax.experimental.pallas import tpu as pltpu
from jax.experimental.pallas import tpu_sc as plsc

BATCH = 4096
HIST = 200
EMB = 64
NCLS = 100
VOCAB = 1000000

NC = 2
NS = 16
NW = NC * NS

RBLK = 32768
RSH = 15
RGRID = (VOCAB + RBLK - 1) // RBLK
TROWS = RGRID * (RBLK // 2)

B_PER_W = BATCH // NW
NBUF = 2
SPLIT0 = 104
SPLIT1 = HIST - SPLIT0


def _relayout_body(i_ref, o_ref):
    t = i_ref[...].T
    o_ref[:, 0:EMB] = t[0:RBLK // 2, :]
    o_ref[:, EMB:2 * EMB] = t[RBLK // 2:, :]


_tc_relayout = pl.pallas_call(
    _relayout_body,
    grid=(RGRID,),
    in_specs=[pl.BlockSpec((EMB, RBLK), lambda i: (0, i))],
    out_specs=pl.BlockSpec((RBLK // 2, 2 * EMB), lambda i: (i, 0)),
    out_shape=jax.ShapeDtypeStruct((TROWS, 2 * EMB), jnp.float32),
)


def _pool_kernel(x_hbm, tab_hbm, out_hbm,
                 jdx_v, pof_v, g0, g1, out_v, s0, s1):
    gbufs = (g0, g1)
    sems = (s0, s1)
    wid = lax.axis_index("s") * NC + lax.axis_index("c")
    base = wid * B_PER_W
    iota = lax.iota(jnp.int32, 16)

    pltpu.sync_copy(x_hbm.at[pl.ds(base * HIST, B_PER_W * HIST)], jdx_v)

    def prep(i, carry):
        raw = jdx_v[pl.ds(i * 16, 16)]
        pof_v[pl.ds(i * 16, 16)] = ((raw >> (RSH - 1)) & 1) * EMB
        jdx_v[pl.ds(i * 16, 16)] = ((raw >> RSH) << (RSH - 1)) + \
            (raw & (RBLK // 2 - 1))
        return carry
    lax.fori_loop(0, B_PER_W * HIST // 16, prep, 0)

    def fire(b, slot):
        off = b * HIST
        pltpu.async_copy(tab_hbm.at[jdx_v.at[pl.ds(off, SPLIT0)]],
                         gbufs[slot].at[pl.ds(0, SPLIT0)], sems[slot])
        pltpu.async_copy(tab_hbm.at[jdx_v.at[pl.ds(off + SPLIT0, SPLIT1)]],
                         gbufs[slot].at[pl.ds(SPLIT0, SPLIT1)], sems[slot])

    def drain(slot):
        pltpu.make_async_copy(tab_hbm.at[pl.ds(0, HIST)], gbufs[slot],
                              sems[slot]).wait()

    def consume(b, slot):
        gb = gbufs[slot]
        off = b * HIST

        def rows(l, acc):
            a0, a1, a2, a3 = acc
            p = plsc.load_gather(pof_v, [jnp.full((16,), off + l, jnp.int32)])
            lsplat = jnp.full((16,), l, jnp.int32)
            col = p + iota
            a0 = a0 + plsc.load_gather(gb, [lsplat, col])
            a1 = a1 + plsc.load_gather(gb, [lsplat, col + 16])
            a2 = a2 + plsc.load_gather(gb, [lsplat, col + 32])
            a3 = a3 + plsc.load_gather(gb, [lsplat, col + 48])
            return (a0, a1, a2, a3)
        zero = jnp.zeros((16,), jnp.float32)
        acc = lax.fori_loop(0, HIST, rows, (zero, zero, zero, zero))

        for c in range(EMB // 16):
            out_v[pl.ds(b * EMB + c * 16, 16)] = acc[c]

    fire(0, 0)

    def group(g, carry):
        for s in range(NBUF):
            b = g * NBUF + s
            nb = b + NBUF - 1
            nslot = (s + NBUF - 1) % NBUF

            @pl.when(nb < B_PER_W)
            def _():
                fire(nb, nslot)

            drain(s)
            consume(b, s)
        return carry
    lax.fori_loop(0, B_PER_W // NBUF, group, 0)

    pltpu.sync_copy(out_v, out_hbm.at[pl.ds(base * EMB, B_PER_W * EMB)])


_pool = functools.partial(
    pl.kernel,
    out_type=jax.ShapeDtypeStruct((BATCH * EMB,), jnp.float32),
    mesh=plsc.VectorSubcoreMesh(core_axis_name="c", subcore_axis_name="s"),
    compiler_params=pltpu.CompilerParams(needs_layout_passes=False),
    scratch_types=[
        pltpu.VMEM((B_PER_W * HIST,), jnp.int32),
        pltpu.VMEM((B_PER_W * HIST,), jnp.int32),
        pltpu.VMEM((HIST, 2 * EMB), jnp.float32),
        pltpu.VMEM((HIST, 2 * EMB), jnp.float32),
        pltpu.VMEM((B_PER_W * EMB,), jnp.float32),
        pltpu.SemaphoreType.DMA,
        pltpu.SemaphoreType.DMA,
    ],
)(_pool_kernel)


def _mm_body(m_ref, x_ref, e0_ref, w_ref, b_ref, o_ref):
    mm = lax.dot_general(
        m_ref[...], w_ref[...], (((1,), (1,)), ((), ())),
        preferred_element_type=jnp.float32)
    e0w = lax.dot_general(
        e0_ref[...], w_ref[...], (((1,), (1,)), ((), ())),
        preferred_element_type=jnp.float32)
    cnt = jnp.sum((x_ref[...] == 0).astype(jnp.float32), axis=1,
                  keepdims=True)
    o_ref[...] = (mm - cnt * e0w) * (1.0 / HIST) + b_ref[...]


def _classify(m, x, e0, W, b):
    return pl.pallas_call(
        _mm_body,
        out_shape=jax.ShapeDtypeStruct((BATCH, NCLS), jnp.float32),
    )(m, x, e0, W, b.reshape(1, NCLS))


def kernel(x, emb, W, b):
    tab = _tc_relayout(emb.T)
    pooled = _pool(x.reshape(-1), tab)
    m = pooled.reshape(BATCH, EMB)
    return _classify(m, x, emb[0:1, :], W, b)
